# Initial kernel scaffold; baseline (speedup 1.0000x reference)
import jax
import jax.numpy as jnp
from jax.experimental import pallas as pl


def kernel(feats, coords, Wq, Wk, Wv, Wo, bo):
    raise NotImplementedError("write your pallas kernel here")


# speedup vs baseline: 5.3893x; 5.3893x over previous
import functools

import jax
import jax.numpy as jnp
from jax import lax
from jax.experimental import pallas as pl
from jax.experimental.pallas import tpu as pltpu
from jax.experimental.pallas import tpu_sc as plsc

N = 4096
C = 512
H = 8
D = 64
K = 16

RB = 256
PB = 128

_NC = 2
_NS = 16
_NW = _NC * _NS


def _knn_proj_body(cn_ref, ct_ref, f_ref, wq_ref, wk_ref, wv_ref,
                   idx_ref, q_ref, k_ref, v_ref):
    cb = cn_ref[...]
    ct = ct_ref[...]
    xb = cb[:, 0:1] * cb[:, 0:1]
    yb = cb[:, 1:2] * cb[:, 1:2]
    zb = cb[:, 2:3] * cb[:, 2:3]
    sq_i = (xb + zb) + yb
    xr = ct[0:1, :] * ct[0:1, :]
    yr = ct[1:2, :] * ct[1:2, :]
    zr = ct[2:3, :] * ct[2:3, :]
    sq_j = (xr + zr) + yr
    g = jnp.dot(cb, ct, preferred_element_type=jnp.float32)
    dist = (sq_i + sq_j) - g * 2.0
    lane = lax.broadcasted_iota(jnp.int32, (RB, N), 1)
    cols = []
    for _ in range(K):
        m = jnp.min(dist, axis=1, keepdims=True)
        cand = jnp.where(dist == m, lane, N)
        sel = jnp.min(cand, axis=1, keepdims=True)
        cols.append(sel)
        dist = jnp.where(lane == sel, jnp.float32(jnp.inf), dist)
    idx_ref[...] = jnp.concatenate(cols, axis=1)
    f = f_ref[...]
    dn = (((1,), (1,)), ((), ()))
    hp = None
    q_ref[...] = lax.dot_general(f, wq_ref[...], dn,
                                 preferred_element_type=jnp.float32,
                                 precision=hp)
    k_ref[...] = lax.dot_general(f, wk_ref[...], dn,
                                 preferred_element_type=jnp.float32,
                                 precision=hp)
    v_ref[...] = lax.dot_general(f, wv_ref[...], dn,
                                 preferred_element_type=jnp.float32,
                                 precision=hp)


def _knn_proj(coords_nat, coords_t, feats, Wq, Wk, Wv):
    nb = N // RB
    return pl.pallas_call(
        _knn_proj_body,
        grid=(nb,),
        in_specs=[
            pl.BlockSpec((RB, 8), lambda i: (i, 0)),
            pl.BlockSpec((8, N), lambda i: (0, 0)),
            pl.BlockSpec((RB, C), lambda i: (i, 0)),
            pl.BlockSpec((C, C), lambda i: (0, 0)),
            pl.BlockSpec((C, C), lambda i: (0, 0)),
            pl.BlockSpec((C, C), lambda i: (0, 0)),
        ],
        out_specs=[
            pl.BlockSpec((RB, K), lambda i: (i, 0)),
            pl.BlockSpec((RB, C), lambda i: (i, 0)),
            pl.BlockSpec((RB, C), lambda i: (i, 0)),
            pl.BlockSpec((RB, C), lambda i: (i, 0)),
        ],
        out_shape=[
            jax.ShapeDtypeStruct((N, K), jnp.int32),
            jax.ShapeDtypeStruct((N, C), jnp.float32),
            jax.ShapeDtypeStruct((N, C), jnp.float32),
            jax.ShapeDtypeStruct((N, C), jnp.float32),
        ],
    )(coords_nat, coords_t, feats, Wq, Wk, Wv)


def _sc_gather(table_k, table_v, idx):
    B = idx.shape[0]
    bpw = B // _NW
    ch = 64
    nch = bpw // ch
    mesh = plsc.VectorSubcoreMesh(core_axis_name="c", subcore_axis_name="s")

    @functools.partial(
        pl.kernel,
        out_type=(jax.ShapeDtypeStruct((B, C), jnp.float32),
                  jax.ShapeDtypeStruct((B, C), jnp.float32)),
        mesh=mesh,
        scratch_types=[
            pltpu.VMEM((bpw,), jnp.int32),
            pltpu.VMEM((ch, C), jnp.float32),
            pltpu.VMEM((ch, C), jnp.float32),
            pltpu.SemaphoreType.DMA,
            pltpu.SemaphoreType.DMA,
        ],
    )
    def gather_kernel(tk_hbm, tv_hbm, idx_hbm, ok_hbm, ov_hbm,
                      idx_v, rk_v, rv_v, sem_k, sem_v):
        wid = lax.axis_index("s") * _NC + lax.axis_index("c")
        base = wid * bpw
        pltpu.sync_copy(idx_hbm.at[pl.ds(base, bpw)], idx_v)

        @pl.loop(0, nch)
        def _(c):
            row0 = base + c * ch
            ick = idx_v.at[pl.ds(c * ch, ch)]
            cp_k = pltpu.async_copy(tk_hbm.at[ick], rk_v, sem_k)
            cp_v = pltpu.async_copy(tv_hbm.at[ick], rv_v, sem_v)
            cp_k.wait()
            cp_v.wait()
            pltpu.sync_copy(rk_v, ok_hbm.at[pl.ds(row0, ch)])
            pltpu.sync_copy(rv_v, ov_hbm.at[pl.ds(row0, ch)])

    return gather_kernel(table_k, table_v, idx)


def _attn_body(q_ref, kn_ref, vn_ref, wo_ref, bo_ref, o_ref):
    seg = (lax.broadcasted_iota(jnp.int32, (C, H), 0) // D ==
           lax.broadcasted_iota(jnp.int32, (C, H), 1)).astype(jnp.float32)
    q = q_ref[...]
    kn = kn_ref[...].reshape(PB, K, C)
    prod = (q[:, None, :] * kn).reshape(PB * K, C)
    s = jnp.dot(prod, seg, preferred_element_type=jnp.float32,
                precision=lax.Precision.HIGHEST) / 8.0
    s3 = s.reshape(PB, K, H)
    m = jnp.max(s3, axis=1, keepdims=True)
    e = jnp.exp(s3 - m)
    den = jnp.sum(e, axis=1, keepdims=True)
    w = (e / den).reshape(PB * K, H)
    wexp = lax.dot_general(w, seg, (((1,), (1,)), ((), ())),
                           preferred_element_type=jnp.float32,
                           precision=lax.Precision.HIGHEST)
    vn = vn_ref[...]
    comb = (wexp * vn).reshape(PB, K, C).sum(axis=1)
    out = lax.dot_general(comb, wo_ref[...], (((1,), (1,)), ((), ())),
                          preferred_element_type=jnp.float32)
    o_ref[...] = out + bo_ref[...]


def _attend(q, kn, vn, Wo, bo2):
    nb = N // PB
    return pl.pallas_call(
        _attn_body,
        grid=(nb,),
        in_specs=[
            pl.BlockSpec((PB, C), lambda i: (i, 0)),
            pl.BlockSpec((PB * K, C), lambda i: (i, 0)),
            pl.BlockSpec((PB * K, C), lambda i: (i, 0)),
            pl.BlockSpec((C, C), lambda i: (0, 0)),
            pl.BlockSpec((1, C), lambda i: (0, 0)),
        ],
        out_specs=pl.BlockSpec((PB, C), lambda i: (i, 0)),
        out_shape=jax.ShapeDtypeStruct((N, C), jnp.float32),
    )(q, kn, vn, Wo, bo2)


def kernel(feats, coords, Wq, Wk, Wv, Wo, bo):
    coords_nat = jnp.pad(coords, ((0, 0), (0, 5)))
    coords_t = coords_nat.T
    knn_idx, q, kf, vf = _knn_proj(coords_nat, coords_t, feats, Wq, Wk, Wv)
    idx_flat = knn_idx.reshape(-1)
    kn, vn = _sc_gather(kf, vf, idx_flat)
    return _attend(q, kn, vn, Wo, bo.reshape(1, C))


# speedup vs baseline: 7.6609x; 1.4215x over previous
import functools

import jax
import jax.numpy as jnp
from jax import lax
from jax.experimental import pallas as pl
from jax.experimental.pallas import tpu as pltpu
from jax.experimental.pallas import tpu_sc as plsc

N = 4096
C = 512
H = 8
D = 64
K = 16

RB = 256
PB = 128

_NC = 2
_NS = 16
_NW = _NC * _NS


def _knn_proj_body(cn_ref, ct_ref, f_ref, wq_ref, wk_ref, wv_ref,
                   idx_ref, q_ref, k_ref, v_ref):
    cb = cn_ref[...]
    ct = ct_ref[...]
    xb = cb[:, 0:1] * cb[:, 0:1]
    yb = cb[:, 1:2] * cb[:, 1:2]
    zb = cb[:, 2:3] * cb[:, 2:3]
    sq_i = (xb + zb) + yb
    xr = ct[0:1, :] * ct[0:1, :]
    yr = ct[1:2, :] * ct[1:2, :]
    zr = ct[2:3, :] * ct[2:3, :]
    sq_j = (xr + zr) + yr
    g = jnp.dot(cb, ct, preferred_element_type=jnp.float32)
    dist = (sq_i + sq_j) - g * 2.0
    lane = lax.broadcasted_iota(jnp.int32, (RB, N), 1)
    cols = []
    for _ in range(K):
        m = jnp.min(dist, axis=1, keepdims=True)
        cand = jnp.where(dist == m, lane, N)
        sel = jnp.min(cand, axis=1, keepdims=True)
        cols.append(sel)
        dist = jnp.where(lane == sel, jnp.float32(jnp.inf), dist)
    idx_ref[...] = jnp.concatenate(cols, axis=1)
    f = f_ref[...]
    dn = (((1,), (1,)), ((), ()))
    hp = None
    q_ref[...] = lax.dot_general(f, wq_ref[...], dn,
                                 preferred_element_type=jnp.float32,
                                 precision=hp)
    k_ref[...] = lax.dot_general(f, wk_ref[...], dn,
                                 preferred_element_type=jnp.float32,
                                 precision=hp)
    v_ref[...] = lax.dot_general(f, wv_ref[...], dn,
                                 preferred_element_type=jnp.float32,
                                 precision=hp)


def _knn_proj(coords_nat, coords_t, feats, Wq, Wk, Wv):
    nb = N // RB
    return pl.pallas_call(
        _knn_proj_body,
        grid=(nb,),
        in_specs=[
            pl.BlockSpec((RB, 8), lambda i: (i, 0)),
            pl.BlockSpec((8, N), lambda i: (0, 0)),
            pl.BlockSpec((RB, C), lambda i: (i, 0)),
            pl.BlockSpec((C, C), lambda i: (0, 0)),
            pl.BlockSpec((C, C), lambda i: (0, 0)),
            pl.BlockSpec((C, C), lambda i: (0, 0)),
        ],
        out_specs=[
            pl.BlockSpec((RB, K), lambda i: (i, 0)),
            pl.BlockSpec((RB, C), lambda i: (i, 0)),
            pl.BlockSpec((RB, C), lambda i: (i, 0)),
            pl.BlockSpec((RB, C), lambda i: (i, 0)),
        ],
        out_shape=[
            jax.ShapeDtypeStruct((N, K), jnp.int32),
            jax.ShapeDtypeStruct((N, C), jnp.float32),
            jax.ShapeDtypeStruct((N, C), jnp.float32),
            jax.ShapeDtypeStruct((N, C), jnp.float32),
        ],
    )(coords_nat, coords_t, feats, Wq, Wk, Wv)


def _sc_gather(table_k, table_v, idx):
    B = idx.shape[0]
    bpw = B // _NW
    ch = 64
    nch = bpw // ch
    mesh = plsc.VectorSubcoreMesh(core_axis_name="c", subcore_axis_name="s")

    @functools.partial(
        pl.kernel,
        out_type=(jax.ShapeDtypeStruct((B, C), jnp.float32),
                  jax.ShapeDtypeStruct((B, C), jnp.float32)),
        mesh=mesh,
        scratch_types=[
            pltpu.VMEM((bpw,), jnp.int32),
            pltpu.VMEM((ch, C), jnp.float32),
            pltpu.VMEM((ch, C), jnp.float32),
            pltpu.SemaphoreType.DMA,
            pltpu.SemaphoreType.DMA,
        ],
    )
    def gather_kernel(tk_hbm, tv_hbm, idx_hbm, ok_hbm, ov_hbm,
                      idx_v, rk_v, rv_v, sem_k, sem_v):
        wid = lax.axis_index("s") * _NC + lax.axis_index("c")
        base = wid * bpw
        pltpu.sync_copy(idx_hbm.at[pl.ds(base, bpw)], idx_v)

        @pl.loop(0, nch)
        def _(c):
            row0 = base + c * ch
            ick = idx_v.at[pl.ds(c * ch, ch)]
            cp_k = pltpu.async_copy(tk_hbm.at[ick], rk_v, sem_k)
            cp_v = pltpu.async_copy(tv_hbm.at[ick], rv_v, sem_v)
            cp_k.wait()
            cp_v.wait()
            pltpu.sync_copy(rk_v, ok_hbm.at[pl.ds(row0, ch)])
            pltpu.sync_copy(rv_v, ov_hbm.at[pl.ds(row0, ch)])

    return gather_kernel(table_k, table_v, idx)


def _attn_body(q_ref, kn_ref, vn_ref, wo_ref, bo_ref, o_ref):
    seg = (lax.broadcasted_iota(jnp.int32, (C, H), 0) // D ==
           lax.broadcasted_iota(jnp.int32, (C, H), 1)).astype(jnp.float32)
    q = q_ref[...]
    kn = kn_ref[...].reshape(PB, K, C)
    prod = (q[:, None, :] * kn).reshape(PB * K, C)
    ph = prod.astype(jnp.bfloat16).astype(jnp.float32)
    plo = prod - ph
    s = (jnp.dot(ph, seg, preferred_element_type=jnp.float32) +
         jnp.dot(plo, seg, preferred_element_type=jnp.float32)) / 8.0
    s3 = s.reshape(PB, K, H)
    m = jnp.max(s3, axis=1, keepdims=True)
    e = jnp.exp(s3 - m)
    den = jnp.sum(e, axis=1, keepdims=True)
    w = (e / den).reshape(PB * K, H)
    wh = w.astype(jnp.bfloat16).astype(jnp.float32)
    wlo = w - wh
    dnt = (((1,), (1,)), ((), ()))
    wexp = (lax.dot_general(wh, seg, dnt,
                            preferred_element_type=jnp.float32) +
            lax.dot_general(wlo, seg, dnt,
                            preferred_element_type=jnp.float32))
    vn = vn_ref[...]
    comb = (wexp * vn).reshape(PB, K, C).sum(axis=1)
    out = lax.dot_general(comb, wo_ref[...], (((1,), (1,)), ((), ())),
                          preferred_element_type=jnp.float32)
    o_ref[...] = out + bo_ref[...]


def _attend(q, kn, vn, Wo, bo2):
    nb = N // PB
    return pl.pallas_call(
        _attn_body,
        grid=(nb,),
        in_specs=[
            pl.BlockSpec((PB, C), lambda i: (i, 0)),
            pl.BlockSpec((PB * K, C), lambda i: (i, 0)),
            pl.BlockSpec((PB * K, C), lambda i: (i, 0)),
            pl.BlockSpec((C, C), lambda i: (0, 0)),
            pl.BlockSpec((1, C), lambda i: (0, 0)),
        ],
        out_specs=pl.BlockSpec((PB, C), lambda i: (i, 0)),
        out_shape=jax.ShapeDtypeStruct((N, C), jnp.float32),
    )(q, kn, vn, Wo, bo2)


def kernel(feats, coords, Wq, Wk, Wv, Wo, bo):
    coords_nat = jnp.pad(coords, ((0, 0), (0, 5)))
    coords_t = coords_nat.T
    knn_idx, q, kf, vf = _knn_proj(coords_nat, coords_t, feats, Wq, Wk, Wv)
    idx_flat = knn_idx.reshape(-1)
    kn, vn = _sc_gather(kf, vf, idx_flat)
    return _attend(q, kn, vn, Wo, bo.reshape(1, C))


# speedup vs baseline: 8.9285x; 1.1655x over previous
import functools

import jax
import jax.numpy as jnp
from jax import lax
from jax.experimental import pallas as pl
from jax.experimental.pallas import tpu as pltpu
from jax.experimental.pallas import tpu_sc as plsc

N = 4096
C = 512
H = 8
D = 64
K = 16

RB = 256
PB = 128

_NC = 2
_NS = 16
_NW = _NC * _NS


def _proj_body(f_ref, wq_ref, wk_ref, wv_ref, q_ref, k_ref, v_ref):
    f = f_ref[...]
    dn = (((1,), (1,)), ((), ()))
    q_ref[...] = lax.dot_general(f, wq_ref[...], dn,
                                 preferred_element_type=jnp.float32)
    k_ref[...] = lax.dot_general(f, wk_ref[...], dn,
                                 preferred_element_type=jnp.float32)
    v_ref[...] = lax.dot_general(f, wv_ref[...], dn,
                                 preferred_element_type=jnp.float32)


def _proj(feats, Wq, Wk, Wv):
    nb = N // RB
    return pl.pallas_call(
        _proj_body,
        grid=(nb,),
        in_specs=[
            pl.BlockSpec((RB, C), lambda i: (i, 0)),
            pl.BlockSpec((C, C), lambda i: (0, 0)),
            pl.BlockSpec((C, C), lambda i: (0, 0)),
            pl.BlockSpec((C, C), lambda i: (0, 0)),
        ],
        out_specs=[
            pl.BlockSpec((RB, C), lambda i: (i, 0)),
            pl.BlockSpec((RB, C), lambda i: (i, 0)),
            pl.BlockSpec((RB, C), lambda i: (i, 0)),
        ],
        out_shape=[
            jax.ShapeDtypeStruct((N, C), jnp.float32),
            jax.ShapeDtypeStruct((N, C), jnp.float32),
            jax.ShapeDtypeStruct((N, C), jnp.float32),
        ],
    )(feats, Wq, Wk, Wv)


def _knn_body(cn_ref, ct_ref, idx_ref):
    cb = cn_ref[...]
    ct = ct_ref[...]
    xb = cb[:, 0:1] * cb[:, 0:1]
    yb = cb[:, 1:2] * cb[:, 1:2]
    zb = cb[:, 2:3] * cb[:, 2:3]
    sq_i = (xb + zb) + yb
    xr = ct[0:1, :] * ct[0:1, :]
    yr = ct[1:2, :] * ct[1:2, :]
    zr = ct[2:3, :] * ct[2:3, :]
    sq_j = (xr + zr) + yr
    g = jnp.dot(cb, ct, preferred_element_type=jnp.float32)
    dist = (sq_i + sq_j) - g * 2.0
    lane = lax.broadcasted_iota(jnp.int32, (RB, N), 1)
    cols = []
    for _ in range(K):
        m = jnp.min(dist, axis=1, keepdims=True)
        cand = jnp.where(dist == m, lane, N)
        sel = jnp.min(cand, axis=1, keepdims=True)
        cols.append(sel)
        dist = jnp.where(lane == sel, jnp.float32(jnp.inf), dist)
    idx_ref[...] = jnp.concatenate(cols, axis=1)


def _knn_half(coords_nat, coords_t, half):
    nb = N // RB // 2
    off = half * nb
    return pl.pallas_call(
        _knn_body,
        grid=(nb,),
        in_specs=[
            pl.BlockSpec((RB, 8), lambda i: (i + off, 0)),
            pl.BlockSpec((8, N), lambda i: (0, 0)),
        ],
        out_specs=pl.BlockSpec((RB, K), lambda i: (i, 0)),
        out_shape=jax.ShapeDtypeStruct((N // 2, K), jnp.int32),
    )(coords_nat, coords_t)


def _sc_gather(table_k, table_v, idx):
    B = idx.shape[0]
    bpw = B // _NW
    ch = 64
    nch = bpw // ch
    mesh = plsc.VectorSubcoreMesh(core_axis_name="c", subcore_axis_name="s")

    @functools.partial(
        pl.kernel,
        out_type=(jax.ShapeDtypeStruct((B, C), jnp.float32),
                  jax.ShapeDtypeStruct((B, C), jnp.float32)),
        mesh=mesh,
        scratch_types=[
            pltpu.VMEM((bpw,), jnp.int32),
            pltpu.VMEM((ch, C), jnp.float32),
            pltpu.VMEM((ch, C), jnp.float32),
            pltpu.SemaphoreType.DMA,
            pltpu.SemaphoreType.DMA,
        ],
    )
    def gather_kernel(tk_hbm, tv_hbm, idx_hbm, ok_hbm, ov_hbm,
                      idx_v, rk_v, rv_v, sem_k, sem_v):
        wid = lax.axis_index("s") * _NC + lax.axis_index("c")
        base = wid * bpw
        pltpu.sync_copy(idx_hbm.at[pl.ds(base, bpw)], idx_v)

        @pl.loop(0, nch)
        def _(c):
            row0 = base + c * ch
            ick = idx_v.at[pl.ds(c * ch, ch)]
            cp_k = pltpu.async_copy(tk_hbm.at[ick], rk_v, sem_k)
            cp_v = pltpu.async_copy(tv_hbm.at[ick], rv_v, sem_v)
            cp_k.wait()
            cp_v.wait()
            pltpu.sync_copy(rk_v, ok_hbm.at[pl.ds(row0, ch)])
            pltpu.sync_copy(rv_v, ov_hbm.at[pl.ds(row0, ch)])

    return gather_kernel(table_k, table_v, idx)


def _attn_body(q_ref, kn_ref, vn_ref, wo_ref, bo_ref, o_ref):
    seg = (lax.broadcasted_iota(jnp.int32, (C, H), 0) // D ==
           lax.broadcasted_iota(jnp.int32, (C, H), 1)).astype(jnp.float32)
    q = q_ref[...]
    kn = kn_ref[...].reshape(PB, K, C)
    prod = (q[:, None, :] * kn).reshape(PB * K, C)
    ph = prod.astype(jnp.bfloat16).astype(jnp.float32)
    plo = prod - ph
    s = (jnp.dot(ph, seg, preferred_element_type=jnp.float32) +
         jnp.dot(plo, seg, preferred_element_type=jnp.float32)) / 8.0
    s3 = s.reshape(PB, K, H)
    m = jnp.max(s3, axis=1, keepdims=True)
    e = jnp.exp(s3 - m)
    den = jnp.sum(e, axis=1, keepdims=True)
    w = (e / den).reshape(PB * K, H)
    wh = w.astype(jnp.bfloat16).astype(jnp.float32)
    wlo = w - wh
    dnt = (((1,), (1,)), ((), ()))
    wexp = (lax.dot_general(wh, seg, dnt,
                            preferred_element_type=jnp.float32) +
            lax.dot_general(wlo, seg, dnt,
                            preferred_element_type=jnp.float32))
    vn = vn_ref[...]
    comb = (wexp * vn).reshape(PB, K, C).sum(axis=1)
    out = lax.dot_general(comb, wo_ref[...], (((1,), (1,)), ((), ())),
                          preferred_element_type=jnp.float32)
    o_ref[...] = out + bo_ref[...]


def _attend(q, kn, vn, Wo, bo2, half):
    nb = N // PB // 2
    qoff = half * nb
    return pl.pallas_call(
        _attn_body,
        grid=(nb,),
        in_specs=[
            pl.BlockSpec((PB, C), lambda i: (i + qoff, 0)),
            pl.BlockSpec((PB * K, C), lambda i: (i, 0)),
            pl.BlockSpec((PB * K, C), lambda i: (i, 0)),
            pl.BlockSpec((C, C), lambda i: (0, 0)),
            pl.BlockSpec((1, C), lambda i: (0, 0)),
        ],
        out_specs=pl.BlockSpec((PB, C), lambda i: (i, 0)),
        out_shape=jax.ShapeDtypeStruct((N // 2, C), jnp.float32),
    )(q, kn, vn, Wo, bo2)


def kernel(feats, coords, Wq, Wk, Wv, Wo, bo):
    coords_nat = jnp.pad(coords, ((0, 0), (0, 5)))
    coords_t = coords_nat.T
    bo2 = bo.reshape(1, C)
    q, kf, vf = _proj(feats, Wq, Wk, Wv)
    idx0 = _knn_half(coords_nat, coords_t, 0).reshape(-1)
    kn0, vn0 = _sc_gather(kf, vf, idx0)
    idx1 = _knn_half(coords_nat, coords_t, 1).reshape(-1)
    kn1, vn1 = _sc_gather(kf, vf, idx1)
    out0 = _attend(q, kn0, vn0, Wo, bo2, 0)
    out1 = _attend(q, kn1, vn1, Wo, bo2, 1)
    return jnp.concatenate([out0, out1], axis=0)
